# baseline (speedup 1.0000x reference)
import functools

import jax
import jax.numpy as jnp
from jax import lax
from jax.experimental import pallas as pl
from jax.experimental.pallas import tpu as pltpu
from jax.experimental.pallas import tpu_sc as plsc

_CHUNK = 32
_NBUF = 4
_GLAG = 2


@functools.lru_cache(maxsize=None)
def _build(N, D, NC, NS):
    NW = NC * NS
    n_per_w = N // NW
    n_chunks = n_per_w // _CHUNK
    mesh = plsc.VectorSubcoreMesh(core_axis_name="c", subcore_axis_name="s")

    @functools.partial(
        pl.kernel,
        mesh=mesh,
        out_type=jax.ShapeDtypeStruct((N, D), jnp.float32),
        scratch_types=(
            [pltpu.VMEM((n_chunks, _CHUNK), jnp.int32)]
            + [pltpu.VMEM((_CHUNK, D), jnp.float32) for _ in range(_NBUF)]
            + [pltpu.SemaphoreType.DMA for _ in range(2 * _NBUF)]
        ),
    )
    def gather_kernel(idx_hbm, table_hbm, out_hbm, idx_v, *bufs_and_sems):
        bufs = bufs_and_sems[:_NBUF]
        gsems = bufs_and_sems[_NBUF:2 * _NBUF]
        osems = bufs_and_sems[2 * _NBUF:]
        wid = lax.axis_index("s") * NC + lax.axis_index("c")
        pltpu.sync_copy(idx_hbm.at[wid], idx_v)
        base = wid * n_per_w

        gather_cp = [None] * _NBUF
        out_cp = [None] * _NBUF
        for j in range(n_chunks + _GLAG):
            if j < n_chunks:
                b = j % _NBUF
                if out_cp[b] is not None:
                    out_cp[b].wait()
                gather_cp[b] = pltpu.async_copy(
                    table_hbm.at[idx_v.at[j]], bufs[b], gsems[b])
            if j >= _GLAG:
                jj = j - _GLAG
                b = jj % _NBUF
                gather_cp[b].wait()
                out_cp[b] = pltpu.async_copy(
                    bufs[b],
                    out_hbm.at[pl.ds(base + jj * _CHUNK, _CHUNK)],
                    osems[b])
        for b in range(_NBUF):
            if out_cp[b] is not None:
                out_cp[b].wait()

    return gather_kernel


def kernel(x, pos_emb):
    B, S = x.shape
    D = pos_emb.shape[1]
    N = B * S
    info = plsc.get_sparse_core_info()
    NC, NS = info.num_cores, info.num_subcores
    NW = NC * NS
    idx = x.reshape(NW, (N // NW) // _CHUNK, _CHUNK).astype(jnp.int32)
    out = _build(N, D, NC, NS)(idx, pos_emb)
    return out.reshape(B, S, D)
